# Initial kernel scaffold; baseline (speedup 1.0000x reference)
import jax
import jax.numpy as jnp
from jax.experimental import pallas as pl


def kernel(node_features, edge_index, W_proj, a_src, a_trg, W_skip, bias):
    raise NotImplementedError("write your pallas kernel here")


# speedup vs baseline: 32.9875x; 32.9875x over previous
import functools

import jax
import jax.numpy as jnp
from jax import lax
from jax.experimental import pallas as pl
from jax.experimental.pallas import tpu as pltpu
from jax.experimental.pallas import tpu_sc as plsc

_N = 10000
_E = 320000
_H = 8
_F = 16
_HF = _H * _F

_TILES = 16
_CORES = 2
_NP = 10112
_ROWS_PT = _NP // _TILES
_BLK = 128
_EP = 323584
_W_BLKS = _EP // (_TILES * _CORES * _BLK)


def _tc_prep_body(x_ref, wp_ref, a2_ref, proj_ref, sc_ref):
    proj = jnp.dot(x_ref[:], wp_ref[:], preferred_element_type=jnp.float32)
    proj_ref[:] = proj
    sc_ref[:] = jnp.dot(proj, a2_ref[:], preferred_element_type=jnp.float32)


_tc_prep = pl.pallas_call(
    _tc_prep_body,
    out_shape=[
        jax.ShapeDtypeStruct((_NP, _HF), jnp.float32),
        jax.ShapeDtypeStruct((_NP, 2 * _F), jnp.float32),
    ],
)


_mesh = plsc.VectorSubcoreMesh(core_axis_name="c", subcore_axis_name="s")


@functools.partial(
    pl.kernel,
    out_type=jax.ShapeDtypeStruct((_CORES, _NP, _F), jnp.float32),
    mesh=_mesh,
    compiler_params=pltpu.CompilerParams(use_tc_tiling_on_sc=False),
    scratch_types=[
        pltpu.VMEM((_BLK,), jnp.int32),
        pltpu.VMEM((_BLK,), jnp.int32),
        pltpu.VMEM((_BLK, _F), jnp.float32),
        pltpu.VMEM((_BLK, _F), jnp.float32),
        pltpu.VMEM((_BLK, _F), jnp.float32),
        pltpu.VMEM_SHARED((_NP, _F), jnp.float32),
        pltpu.SemaphoreType.DMA,
    ],
)
def _sc_den(src_hbm, trg_hbm, ssrc_hbm, strg_hbm, zd_hbm,
            den_hbm, src_v, trg_v, ssrc_v, strg_v, att_v, den_sp, sem):
    c = lax.axis_index("c")
    s = lax.axis_index("s")
    w = s * _CORES + c
    r0 = s * _ROWS_PT

    pltpu.sync_copy(zd_hbm.at[pl.ds(r0, _ROWS_PT)],
                    den_sp.at[pl.ds(r0, _ROWS_PT)])
    plsc.subcore_barrier()

    @pl.loop(0, _W_BLKS)
    def _blk(j):
        base = w * (_W_BLKS * _BLK) + j * _BLK
        pltpu.sync_copy(src_hbm.at[pl.ds(base, _BLK)], src_v)
        pltpu.sync_copy(trg_hbm.at[pl.ds(base, _BLK)], trg_v)
        d1 = pltpu.async_copy(ssrc_hbm.at[src_v], ssrc_v, sem)
        d2 = pltpu.async_copy(strg_hbm.at[trg_v], strg_v, sem)
        d1.wait()
        d2.wait()

        @pl.loop(0, _BLK)
        def _edge(r):
            sco = ssrc_v[r, :] + strg_v[r, :]
            sco = jnp.maximum(sco, 0.2 * sco)
            att_v[r, :] = jnp.exp(sco)

        pltpu.sync_copy(att_v, den_sp.at[trg_v], add=True)

    plsc.subcore_barrier()
    pltpu.sync_copy(den_sp.at[pl.ds(r0, _ROWS_PT)],
                    den_hbm.at[c, pl.ds(r0, _ROWS_PT)])


@functools.partial(
    pl.kernel,
    out_type=jax.ShapeDtypeStruct((_CORES, _NP, _HF), jnp.float32),
    mesh=_mesh,
    compiler_params=pltpu.CompilerParams(use_tc_tiling_on_sc=False),
    scratch_types=[
        pltpu.VMEM((_BLK,), jnp.int32),
        pltpu.VMEM((_BLK,), jnp.int32),
        pltpu.VMEM((_BLK, _F), jnp.float32),
        pltpu.VMEM((_BLK, _F), jnp.float32),
        pltpu.VMEM((_BLK, _HF), jnp.float32),
        pltpu.VMEM((_BLK, _HF), jnp.float32),
        pltpu.VMEM_SHARED((_NP, _HF), jnp.float32),
        pltpu.SemaphoreType.DMA,
    ],
)
def _sc_agg(src_hbm, trg_hbm, ssrc_hbm, strg_hbm, proj_hbm, zo_hbm,
            out_hbm, src_v, trg_v, ssrc_v, strg_v, proj_v, wgt_v,
            out_sp, sem):
    c = lax.axis_index("c")
    s = lax.axis_index("s")
    w = s * _CORES + c
    r0 = s * _ROWS_PT

    pltpu.sync_copy(zo_hbm.at[pl.ds(r0, _ROWS_PT)],
                    out_sp.at[pl.ds(r0, _ROWS_PT)])
    plsc.subcore_barrier()

    @pl.loop(0, _W_BLKS)
    def _blk(j):
        base = w * (_W_BLKS * _BLK) + j * _BLK
        pltpu.sync_copy(src_hbm.at[pl.ds(base, _BLK)], src_v)
        pltpu.sync_copy(trg_hbm.at[pl.ds(base, _BLK)], trg_v)
        d1 = pltpu.async_copy(ssrc_hbm.at[src_v], ssrc_v, sem)
        d2 = pltpu.async_copy(strg_hbm.at[trg_v], strg_v, sem)
        d3 = pltpu.async_copy(proj_hbm.at[src_v], proj_v, sem)
        d1.wait()
        d2.wait()
        d3.wait()

        @pl.loop(0, _BLK)
        def _edge(r):
            sco = ssrc_v[r, :] + strg_v[r, :]
            sco = jnp.maximum(sco, 0.2 * sco)
            p = jnp.exp(sco)
            for h in range(_H):
                wgt_v[r, pl.ds(h * _F, _F)] = (
                    proj_v[r, pl.ds(h * _F, _F)] * p[h])

        pltpu.sync_copy(wgt_v, out_sp.at[trg_v], add=True)

    plsc.subcore_barrier()
    pltpu.sync_copy(out_sp.at[pl.ds(r0, _ROWS_PT)],
                    out_hbm.at[c, pl.ds(r0, _ROWS_PT)])


def _tc_fin_body(p_ref, d_ref, e_ref, x_ref, ws_ref, b_ref, o_ref):
    agg = p_ref[0] + p_ref[1]
    den = d_ref[0] + d_ref[1]
    recip = 1.0 / (den + 1e-16)
    recip128 = jnp.dot(recip, e_ref[:], preferred_element_type=jnp.float32)
    acc = agg * recip128
    acc = acc + jnp.dot(x_ref[:], ws_ref[:], preferred_element_type=jnp.float32)
    acc = acc + b_ref[:]
    o_ref[:] = jnp.where(acc > 0, acc, jnp.exp(jnp.minimum(acc, 0.0)) - 1.0)


_tc_fin = pl.pallas_call(
    _tc_fin_body,
    out_shape=jax.ShapeDtypeStruct((_NP, _HF), jnp.float32),
)


def kernel(node_features, edge_index, W_proj, a_src, a_trg, W_skip, bias):
    x = node_features.astype(jnp.float32)
    xp = jnp.pad(x, ((0, _NP - _N), (0, 0)))

    src = edge_index[0].astype(jnp.int32)
    trg = edge_index[1].astype(jnp.int32)
    pad_e = _EP - _E
    src_p = jnp.concatenate([src, jnp.full((pad_e,), _N, jnp.int32)])
    trg_p = jnp.concatenate([trg, jnp.full((pad_e,), _N, jnp.int32)])

    rows = jnp.arange(_HF, dtype=jnp.int32)
    hcol = rows // _F
    m_src = jnp.zeros((_HF, _H), jnp.float32).at[rows, hcol].set(
        a_src.reshape(_HF).astype(jnp.float32))
    m_trg = jnp.zeros((_HF, _H), jnp.float32).at[rows, hcol].set(
        a_trg.reshape(_HF).astype(jnp.float32))
    a2 = jnp.concatenate([m_src, m_src, m_trg, m_trg], axis=1)

    proj, scores = _tc_prep(xp, W_proj.astype(jnp.float32), a2)
    ssrc_tab = scores[:, :_F]
    strg_tab = scores[:, _F:]

    zo = jnp.zeros((_NP, _HF), jnp.float32)
    zd = jnp.zeros((_NP, _F), jnp.float32)
    dens = _sc_den(src_p, trg_p, ssrc_tab, strg_tab, zd)
    partials = _sc_agg(src_p, trg_p, ssrc_tab, strg_tab, proj, zo)

    e_mat = jnp.concatenate([
        (hcol[None, :] == jnp.arange(_H, dtype=jnp.int32)[:, None]
         ).astype(jnp.float32),
        jnp.zeros((_H, _HF), jnp.float32),
    ], axis=0)
    out = _tc_fin(partials, dens, e_mat, xp, W_skip.astype(jnp.float32),
                  bias.reshape(1, _HF).astype(jnp.float32))
    return out[:_N]


# speedup vs baseline: 36.3801x; 1.1028x over previous
import functools

import jax
import jax.numpy as jnp
from jax import lax
from jax.experimental import pallas as pl
from jax.experimental.pallas import tpu as pltpu
from jax.experimental.pallas import tpu_sc as plsc

_N = 10000
_E = 320000
_H = 8
_F = 16
_HF = _H * _F

_TILES = 16
_CORES = 2
_NP = 10112
_ROWS_PT = _NP // _TILES
_BLK = 128
_NBLK = 80
_EP = _TILES * _CORES * _NBLK * _BLK
_SHIFT = 14
_MASK = (1 << _SHIFT) - 1


def _tc_prep_body(x_ref, wp_ref, a2_ref, proj_ref, sc_ref):
    proj = jnp.dot(x_ref[:], wp_ref[:], preferred_element_type=jnp.float32)
    proj_ref[:] = proj
    sc_ref[:] = jnp.dot(proj, a2_ref[:], preferred_element_type=jnp.float32)


_tc_prep = pl.pallas_call(
    _tc_prep_body,
    out_shape=[
        jax.ShapeDtypeStruct((_NP, _HF), jnp.float32),
        jax.ShapeDtypeStruct((_NP, _F), jnp.float32),
    ],
)


_mesh = plsc.VectorSubcoreMesh(core_axis_name="c", subcore_axis_name="s")


def _unpack_idx(pk_i, src_i, trg_i):
    @pl.loop(0, _BLK // 16)
    def _u(q):
        v = pk_i[pl.ds(q * 16, 16)]
        src_i[pl.ds(q * 16, 16)] = jnp.bitwise_and(v, _MASK)
        trg_i[pl.ds(q * 16, 16)] = jnp.right_shift(v, _SHIFT)


def _edge_p(a_row, t_row):
    lane = lax.iota(jnp.int32, 16)
    sel = jnp.where(lane < _H, a_row, t_row)
    sco = sel + lax.rev(sel, dimensions=(0,))
    sco = jnp.maximum(sco, 0.2 * sco)
    return jnp.exp(sco)


@functools.partial(
    pl.kernel,
    out_type=jax.ShapeDtypeStruct((_CORES, _NP, _F), jnp.float32),
    mesh=_mesh,
    compiler_params=pltpu.CompilerParams(use_tc_tiling_on_sc=False),
    scratch_types=[
        pltpu.VMEM((_BLK,), jnp.int32),
        pltpu.VMEM((_BLK,), jnp.int32),
        pltpu.VMEM((_BLK,), jnp.int32),
        pltpu.VMEM((_BLK, _F), jnp.float32),
        pltpu.VMEM((_BLK, _F), jnp.float32),
        pltpu.VMEM((_BLK, _F), jnp.float32),
        pltpu.VMEM_SHARED((_NP, _F), jnp.float32),
        pltpu.SemaphoreType.DMA,
    ],
)
def _sc_den(epk_hbm, stab_hbm, zd_hbm,
            den_hbm, pk_i, src_i, trg_i, sa_v, st_v, att_v,
            den_sp, gsem):
    c = lax.axis_index("c")
    s = lax.axis_index("s")
    w = s * _CORES + c
    r0 = s * _ROWS_PT
    e0 = w * (_NBLK * _BLK)

    pltpu.sync_copy(zd_hbm.at[pl.ds(r0, _ROWS_PT)],
                    den_sp.at[pl.ds(r0, _ROWS_PT)])
    plsc.subcore_barrier()

    @pl.loop(0, _NBLK)
    def _blk(j):
        pltpu.sync_copy(epk_hbm.at[pl.ds(e0 + j * _BLK, _BLK)], pk_i)
        _unpack_idx(pk_i, src_i, trg_i)
        d1 = pltpu.async_copy(stab_hbm.at[src_i], sa_v, gsem)
        d2 = pltpu.async_copy(stab_hbm.at[trg_i], st_v, gsem)
        d1.wait()
        d2.wait()

        @pl.loop(0, _BLK)
        def _edge(r):
            att_v[r, :] = _edge_p(sa_v[r, :], st_v[r, :])

        pltpu.sync_copy(att_v, den_sp.at[trg_i], add=True)

    plsc.subcore_barrier()
    pltpu.sync_copy(den_sp.at[pl.ds(r0, _ROWS_PT)],
                    den_hbm.at[c, pl.ds(r0, _ROWS_PT)])


@functools.partial(
    pl.kernel,
    out_type=jax.ShapeDtypeStruct((_CORES, _NP, _HF), jnp.float32),
    mesh=_mesh,
    compiler_params=pltpu.CompilerParams(use_tc_tiling_on_sc=False),
    scratch_types=[
        pltpu.VMEM((_BLK,), jnp.int32),
        pltpu.VMEM((_BLK,), jnp.int32),
        pltpu.VMEM((_BLK,), jnp.int32),
        pltpu.VMEM((_BLK, _F), jnp.float32),
        pltpu.VMEM((_BLK, _F), jnp.float32),
        pltpu.VMEM((_BLK, _HF), jnp.float32),
        pltpu.VMEM_SHARED((_NP, _HF), jnp.float32),
        pltpu.SemaphoreType.DMA,
    ],
)
def _sc_agg(epk_hbm, stab_hbm, proj_hbm, zo_hbm,
            out_hbm, pk_i, src_i, trg_i, sa_v, st_v, proj_v,
            out_sp, gsem):
    c = lax.axis_index("c")
    s = lax.axis_index("s")
    w = s * _CORES + c
    r0 = s * _ROWS_PT
    e0 = w * (_NBLK * _BLK)

    pltpu.sync_copy(zo_hbm.at[pl.ds(r0, _ROWS_PT)],
                    out_sp.at[pl.ds(r0, _ROWS_PT)])
    plsc.subcore_barrier()

    @pl.loop(0, _NBLK)
    def _blk(j):
        pltpu.sync_copy(epk_hbm.at[pl.ds(e0 + j * _BLK, _BLK)], pk_i)
        _unpack_idx(pk_i, src_i, trg_i)
        d1 = pltpu.async_copy(stab_hbm.at[src_i], sa_v, gsem)
        d2 = pltpu.async_copy(stab_hbm.at[trg_i], st_v, gsem)
        d3 = pltpu.async_copy(proj_hbm.at[src_i], proj_v, gsem)
        d1.wait()
        d2.wait()
        d3.wait()

        @pl.loop(0, _BLK)
        def _edge(r):
            p = _edge_p(sa_v[r, :], st_v[r, :])
            for h in range(_H):
                proj_v[r, pl.ds(h * _F, _F)] = (
                    proj_v[r, pl.ds(h * _F, _F)] * p[h])

        pltpu.sync_copy(proj_v, out_sp.at[trg_i], add=True)

    plsc.subcore_barrier()
    pltpu.sync_copy(out_sp.at[pl.ds(r0, _ROWS_PT)],
                    out_hbm.at[c, pl.ds(r0, _ROWS_PT)])


def _tc_fin_body(p_ref, d_ref, e_ref, x_ref, ws_ref, b_ref, o_ref):
    agg = p_ref[0] + p_ref[1]
    den = d_ref[0] + d_ref[1]
    recip = 1.0 / (den + 1e-16)
    recip128 = jnp.dot(recip, e_ref[:], preferred_element_type=jnp.float32)
    acc = agg * recip128
    acc = acc + jnp.dot(x_ref[:], ws_ref[:], preferred_element_type=jnp.float32)
    acc = acc + b_ref[:]
    o_ref[:] = jnp.where(acc > 0, acc, jnp.exp(jnp.minimum(acc, 0.0)) - 1.0)


_tc_fin = pl.pallas_call(
    _tc_fin_body,
    out_shape=jax.ShapeDtypeStruct((_NP, _HF), jnp.float32),
)


def kernel(node_features, edge_index, W_proj, a_src, a_trg, W_skip, bias):
    x = node_features.astype(jnp.float32)
    xp = jnp.pad(x, ((0, _NP - _N), (0, 0)))

    src = edge_index[0].astype(jnp.int32)
    trg = edge_index[1].astype(jnp.int32)
    pad_e = _EP - _E
    src_p = jnp.concatenate([src, jnp.full((pad_e,), _N, jnp.int32)])
    trg_p = jnp.concatenate([trg, jnp.full((pad_e,), _N, jnp.int32)])
    epk = src_p + (trg_p << _SHIFT)

    rows = jnp.arange(_HF, dtype=jnp.int32)
    hcol = rows // _F
    m_src = jnp.zeros((_HF, _H), jnp.float32).at[rows, hcol].set(
        a_src.reshape(_HF).astype(jnp.float32))
    m_trg = jnp.zeros((_HF, _H), jnp.float32).at[rows, hcol].set(
        a_trg.reshape(_HF).astype(jnp.float32))
    a2 = jnp.concatenate([m_src, m_trg[:, ::-1]], axis=1)

    proj, stab = _tc_prep(xp, W_proj.astype(jnp.float32), a2)

    zo = jnp.zeros((_NP, _HF), jnp.float32)
    zd = jnp.zeros((_NP, _F), jnp.float32)
    dens = _sc_den(epk, stab, zd)
    partials = _sc_agg(epk, stab, proj, zo)

    e_mat = jnp.concatenate([
        (hcol[None, :] == jnp.arange(_H, dtype=jnp.int32)[:, None]
         ).astype(jnp.float32),
        jnp.zeros((_H, _HF), jnp.float32),
    ], axis=0)
    out = _tc_fin(partials, dens, e_mat, xp, W_skip.astype(jnp.float32),
                  bias.reshape(1, _HF).astype(jnp.float32))
    return out[:_N]


# speedup vs baseline: 44.3719x; 1.2197x over previous
import functools

import jax
import jax.numpy as jnp
from jax import lax
from jax.experimental import pallas as pl
from jax.experimental.pallas import tpu as pltpu
from jax.experimental.pallas import tpu_sc as plsc

_N = 10000
_E = 320000
_H = 8
_F = 16
_HF = _H * _F

_TILES = 16
_CORES = 2
_NP = 10112
_ROWS_PT = _NP // _TILES
_BLK = 128
_IC = 10
_NCH = 8
_NBLK = _IC * _NCH
_EP = _TILES * _CORES * _NBLK * _BLK
_SHIFT = 14
_MASK = (1 << _SHIFT) - 1


def _tc_prep_body(x_ref, wp_ref, a2_ref, proj_ref, sc_ref):
    proj = jnp.dot(x_ref[:], wp_ref[:], preferred_element_type=jnp.float32)
    proj_ref[:] = proj
    sc_ref[:] = jnp.dot(proj, a2_ref[:], preferred_element_type=jnp.float32)


_tc_prep = pl.pallas_call(
    _tc_prep_body,
    out_shape=[
        jax.ShapeDtypeStruct((_NP, _HF), jnp.float32),
        jax.ShapeDtypeStruct((_NP, _F), jnp.float32),
    ],
)


_mesh = plsc.VectorSubcoreMesh(core_axis_name="c", subcore_axis_name="s")


def _edge_p(a_row, t_row):
    lane = lax.iota(jnp.int32, 16)
    sel = jnp.where(lane < _H, a_row, t_row)
    sco = sel + lax.rev(sel, dimensions=(0,))
    sco = jnp.maximum(sco, 0.2 * sco)
    return jnp.exp(sco)


@functools.partial(
    pl.kernel,
    out_type=[
        jax.ShapeDtypeStruct((_CORES, _NP, _HF), jnp.float32),
        jax.ShapeDtypeStruct((_CORES, _NP, _F), jnp.float32),
    ],
    mesh=_mesh,
    compiler_params=pltpu.CompilerParams(use_tc_tiling_on_sc=False),
    scratch_types=[
        pltpu.VMEM((_IC * _BLK,), jnp.int32),
        pltpu.VMEM((_BLK,), jnp.int32),
        pltpu.VMEM((_BLK,), jnp.int32),
        pltpu.VMEM((_BLK, _F), jnp.float32),
        pltpu.VMEM((_BLK, _F), jnp.float32),
        pltpu.VMEM((_BLK, _F), jnp.float32),
        pltpu.VMEM((_BLK, _HF), jnp.float32),
        pltpu.VMEM_SHARED((_NP, _HF), jnp.float32),
        pltpu.VMEM_SHARED((_NP, _F), jnp.float32),
        pltpu.SemaphoreType.DMA,
    ],
)
def _sc_edges(epk_hbm, stab_hbm, proj_hbm, zo_hbm, zd_hbm,
              out_hbm, den_hbm, pk_i, src_i, trg_i, sa_v, st_v, att_v,
              proj_v, out_sp, den_sp, gsem):
    c = lax.axis_index("c")
    s = lax.axis_index("s")
    w = s * _CORES + c
    r0 = s * _ROWS_PT
    e0 = w * (_NBLK * _BLK)

    pltpu.sync_copy(zo_hbm.at[pl.ds(r0, _ROWS_PT)],
                    out_sp.at[pl.ds(r0, _ROWS_PT)])
    pltpu.sync_copy(zd_hbm.at[pl.ds(r0, _ROWS_PT)],
                    den_sp.at[pl.ds(r0, _ROWS_PT)])
    plsc.subcore_barrier()

    @pl.loop(0, _NCH)
    def _chunk(u):
        pltpu.sync_copy(
            epk_hbm.at[pl.ds(e0 + u * (_IC * _BLK), _IC * _BLK)], pk_i)

        @pl.loop(0, _IC)
        def _blk(t):
            @pl.loop(0, _BLK // 16)
            def _u(q):
                v = pk_i[pl.ds(t * _BLK + q * 16, 16)]
                src_i[pl.ds(q * 16, 16)] = jnp.bitwise_and(v, _MASK)
                trg_i[pl.ds(q * 16, 16)] = jnp.right_shift(v, _SHIFT)

            d1 = pltpu.async_copy(stab_hbm.at[src_i], sa_v, gsem)
            d2 = pltpu.async_copy(stab_hbm.at[trg_i], st_v, gsem)
            d3 = pltpu.async_copy(proj_hbm.at[src_i], proj_v, gsem)
            d1.wait()
            d2.wait()
            d3.wait()

            @pl.loop(0, _BLK)
            def _edge(r):
                p = _edge_p(sa_v[r, :], st_v[r, :])
                att_v[r, :] = p
                for h in range(_H):
                    proj_v[r, pl.ds(h * _F, _F)] = (
                        proj_v[r, pl.ds(h * _F, _F)] * p[h])

            pltpu.sync_copy(att_v, den_sp.at[trg_i], add=True)
            pltpu.sync_copy(proj_v, out_sp.at[trg_i], add=True)

    plsc.subcore_barrier()
    pltpu.sync_copy(out_sp.at[pl.ds(r0, _ROWS_PT)],
                    out_hbm.at[c, pl.ds(r0, _ROWS_PT)])
    pltpu.sync_copy(den_sp.at[pl.ds(r0, _ROWS_PT)],
                    den_hbm.at[c, pl.ds(r0, _ROWS_PT)])


def _tc_fin_body(p_ref, d_ref, e_ref, x_ref, ws_ref, b_ref, o_ref):
    agg = p_ref[0] + p_ref[1]
    den = d_ref[0] + d_ref[1]
    recip = 1.0 / (den + 1e-16)
    recip128 = jnp.dot(recip, e_ref[:], preferred_element_type=jnp.float32)
    acc = agg * recip128
    acc = acc + jnp.dot(x_ref[:], ws_ref[:], preferred_element_type=jnp.float32)
    acc = acc + b_ref[:]
    o_ref[:] = jnp.where(acc > 0, acc, jnp.exp(jnp.minimum(acc, 0.0)) - 1.0)


_tc_fin = pl.pallas_call(
    _tc_fin_body,
    out_shape=jax.ShapeDtypeStruct((_NP, _HF), jnp.float32),
)


def kernel(node_features, edge_index, W_proj, a_src, a_trg, W_skip, bias):
    x = node_features.astype(jnp.float32)
    xp = jnp.pad(x, ((0, _NP - _N), (0, 0)))

    src = edge_index[0].astype(jnp.int32)
    trg = edge_index[1].astype(jnp.int32)
    pad_e = _EP - _E
    src_p = jnp.concatenate([src, jnp.full((pad_e,), _N, jnp.int32)])
    trg_p = jnp.concatenate([trg, jnp.full((pad_e,), _N, jnp.int32)])
    epk = src_p + (trg_p << _SHIFT)

    rows = jnp.arange(_HF, dtype=jnp.int32)
    hcol = rows // _F
    m_src = jnp.zeros((_HF, _H), jnp.float32).at[rows, hcol].set(
        a_src.reshape(_HF).astype(jnp.float32))
    m_trg = jnp.zeros((_HF, _H), jnp.float32).at[rows, hcol].set(
        a_trg.reshape(_HF).astype(jnp.float32))
    a2 = jnp.concatenate([m_src, m_trg[:, ::-1]], axis=1)

    proj, stab = _tc_prep(xp, W_proj.astype(jnp.float32), a2)

    zo = jnp.zeros((_NP, _HF), jnp.float32)
    zd = jnp.zeros((_NP, _F), jnp.float32)
    partials, dens = _sc_edges(epk, stab, proj, zo, zd)

    e_mat = jnp.concatenate([
        (hcol[None, :] == jnp.arange(_H, dtype=jnp.int32)[:, None]
         ).astype(jnp.float32),
        jnp.zeros((_H, _HF), jnp.float32),
    ], axis=0)
    out = _tc_fin(partials, dens, e_mat, xp, W_skip.astype(jnp.float32),
                  bias.reshape(1, _HF).astype(jnp.float32))
    return out[:_N]


# speedup vs baseline: 45.8843x; 1.0341x over previous
import functools

import jax
import jax.numpy as jnp
from jax import lax
from jax.experimental import pallas as pl
from jax.experimental.pallas import tpu as pltpu
from jax.experimental.pallas import tpu_sc as plsc

_N = 10000
_E = 320000
_H = 8
_F = 16
_HF = _H * _F

_TILES = 16
_CORES = 2
_NP = 10112
_ROWS_PT = _NP // _TILES
_NPS = 10016
_RPS = _NPS // _TILES
_BLK = 128
_IC = 10
_NBLK = 80
_EP = _TILES * _CORES * _NBLK * _BLK
_SHIFT = 14
_MASK = (1 << _SHIFT) - 1


def _tc_prep_body(x_ref, wp_ref, a2_ref, proj_ref, sc_ref):
    proj = jnp.dot(x_ref[:], wp_ref[:], preferred_element_type=jnp.float32)
    proj_ref[:] = proj
    sc_ref[:] = jnp.dot(proj, a2_ref[:], preferred_element_type=jnp.float32)


_tc_prep = pl.pallas_call(
    _tc_prep_body,
    out_shape=[
        jax.ShapeDtypeStruct((_NP, _HF), jnp.float32),
        jax.ShapeDtypeStruct((_NP, _F), jnp.float32),
    ],
)


_mesh = plsc.VectorSubcoreMesh(core_axis_name="c", subcore_axis_name="s")


def _edge_p(a_row, t_row):
    lane = lax.iota(jnp.int32, 16)
    sel = jnp.where(lane < _H, a_row, t_row)
    sco = sel + lax.rev(sel, dimensions=(0,))
    sco = jnp.maximum(sco, 0.2 * sco)
    return jnp.exp(sco)


@functools.partial(
    pl.kernel,
    out_type=[
        jax.ShapeDtypeStruct((_CORES, _NPS, _HF), jnp.float32),
        jax.ShapeDtypeStruct((_CORES, _NPS, _F), jnp.float32),
    ],
    mesh=_mesh,
    compiler_params=pltpu.CompilerParams(use_tc_tiling_on_sc=False),
    scratch_types=[
        pltpu.VMEM((_IC * _BLK,), jnp.int32),
        pltpu.VMEM((_BLK,), jnp.int32),
        pltpu.VMEM((_BLK,), jnp.int32),
        pltpu.VMEM((_BLK,), jnp.int32),
        pltpu.VMEM((_BLK,), jnp.int32),
        pltpu.VMEM((_BLK, _F), jnp.float32),
        pltpu.VMEM((_BLK, _F), jnp.float32),
        pltpu.VMEM((_BLK, _F), jnp.float32),
        pltpu.VMEM((_BLK, _F), jnp.float32),
        pltpu.VMEM((_BLK, _F), jnp.float32),
        pltpu.VMEM((_BLK, _F), jnp.float32),
        pltpu.VMEM((_BLK, _HF), jnp.float32),
        pltpu.SemaphoreType.DMA,
        pltpu.SemaphoreType.DMA,
        pltpu.SemaphoreType.DMA,
        pltpu.SemaphoreType.DMA,
        pltpu.SemaphoreType.DMA,
        pltpu.VMEM_SHARED((_NPS, _HF), jnp.float32),
        pltpu.VMEM_SHARED((_NPS, _F), jnp.float32),
    ],
)
def _sc_edges(epk_hbm, stab_hbm, proj_hbm, zo_hbm,
              out_hbm, den_hbm, pk_i, si0, si1, ti0, ti1, sa0, sa1,
              st0, st1, at0, at1, pj, g0, g1, psem, s0, s1,
              out_sp, den_sp):
    c = lax.axis_index("c")
    s = lax.axis_index("s")
    w = s * _CORES + c
    r0 = s * _RPS
    e0 = w * (_NBLK * _BLK)

    src_i = (si0, si1)
    trg_i = (ti0, ti1)
    sa_v = (sa0, sa1)
    st_v = (st0, st1)
    att_v = (at0, at1)
    gsem = (g0, g1)
    ssem = (s0, s1)

    def load_chunk(j):
        pltpu.sync_copy(
            epk_hbm.at[pl.ds(e0 + j * _BLK, _IC * _BLK)], pk_i)

    def unpack(toff, b):
        @pl.loop(0, _BLK // 16)
        def _u(q):
            v = pk_i[pl.ds(toff * _BLK + q * 16, 16)]
            src_i[b][pl.ds(q * 16, 16)] = jnp.bitwise_and(v, _MASK)
            trg_i[b][pl.ds(q * 16, 16)] = jnp.right_shift(v, _SHIFT)

    def fire_g(b):
        pltpu.async_copy(stab_hbm.at[src_i[b]], sa_v[b], gsem[b])
        pltpu.async_copy(stab_hbm.at[trg_i[b]], st_v[b], gsem[b])

    def wait_g(b):
        pltpu.make_async_copy(stab_hbm.at[src_i[b]], sa_v[b], gsem[b]).wait()
        pltpu.make_async_copy(stab_hbm.at[trg_i[b]], st_v[b], gsem[b]).wait()

    def fire_pj(b):
        pltpu.async_copy(proj_hbm.at[src_i[b]], pj, psem)

    def wait_pj(b):
        pltpu.make_async_copy(proj_hbm.at[src_i[b]], pj, psem).wait()

    def fire_s(b):
        pltpu.async_copy(att_v[b], den_sp.at[trg_i[b]], ssem[b], add=True)
        pltpu.async_copy(pj, out_sp.at[trg_i[b]], ssem[b], add=True)

    def wait_s(b):
        pltpu.make_async_copy(att_v[b], den_sp.at[trg_i[b]], ssem[b]).wait()
        pltpu.make_async_copy(
            pj, out_sp.at[trg_i[b]], ssem[b]).wait()

    def compute(b):
        @pl.loop(0, _BLK)
        def _edge(r):
            p = _edge_p(sa_v[b][r, :], st_v[b][r, :])
            att_v[b][r, :] = p
            for h in range(_H):
                pj[r, pl.ds(h * _F, _F)] = (
                    pj[r, pl.ds(h * _F, _F)] * p[h])

    pltpu.sync_copy(zo_hbm.at[pl.ds(r0, _RPS)],
                    out_sp.at[pl.ds(r0, _RPS)])
    pltpu.sync_copy(zo_hbm.at[pl.ds(r0, _RPS), pl.ds(0, _F)],
                    den_sp.at[pl.ds(r0, _RPS)])
    load_chunk(0)
    unpack(0, 0)
    fire_g(0)
    plsc.subcore_barrier()

    @pl.loop(0, _NBLK // 2)
    def _pair(k):
        for b in (0, 1):
            j = 2 * k + b
            wait_g(b)

            @pl.when(j >= 1)
            def _():
                wait_s(1 - b)

            fire_pj(b)

            @pl.when(j < _NBLK - 1)
            def _():
                jn = j + 1
                tn = lax.rem(jn, _IC)

                @pl.when(tn == 0)
                def _():
                    load_chunk(jn)

                unpack(tn, 1 - b)
                fire_g(1 - b)

            wait_pj(b)
            compute(b)
            fire_s(b)

    wait_s(1)
    plsc.subcore_barrier()
    pltpu.sync_copy(out_sp.at[pl.ds(r0, _RPS)],
                    out_hbm.at[c, pl.ds(r0, _RPS)])
    pltpu.sync_copy(den_sp.at[pl.ds(r0, _RPS)],
                    den_hbm.at[c, pl.ds(r0, _RPS)])


def _tc_fin_body(p_ref, d_ref, e_ref, x_ref, ws_ref, b_ref, o_ref):
    agg = p_ref[0] + p_ref[1]
    den = d_ref[0] + d_ref[1]
    recip = 1.0 / (den + 1e-16)
    recip128 = jnp.dot(recip, e_ref[:], preferred_element_type=jnp.float32)
    acc = agg * recip128
    acc = acc + jnp.dot(x_ref[:], ws_ref[:], preferred_element_type=jnp.float32)
    acc = acc + b_ref[:]
    o_ref[:] = jnp.where(acc > 0, acc, jnp.exp(jnp.minimum(acc, 0.0)) - 1.0)


_tc_fin = pl.pallas_call(
    _tc_fin_body,
    out_shape=jax.ShapeDtypeStruct((_NPS, _HF), jnp.float32),
)


def kernel(node_features, edge_index, W_proj, a_src, a_trg, W_skip, bias):
    x = node_features.astype(jnp.float32)
    xp = jnp.pad(x, ((0, _NP - _N), (0, 0)))

    src = edge_index[0].astype(jnp.int32)
    trg = edge_index[1].astype(jnp.int32)
    pad_e = _EP - _E
    src_p = jnp.concatenate([src, jnp.full((pad_e,), _N, jnp.int32)])
    trg_p = jnp.concatenate([trg, jnp.full((pad_e,), _N, jnp.int32)])
    epk = src_p + (trg_p << _SHIFT)

    rows = jnp.arange(_HF, dtype=jnp.int32)
    hcol = rows // _F
    m_src = jnp.zeros((_HF, _H), jnp.float32).at[rows, hcol].set(
        a_src.reshape(_HF).astype(jnp.float32))
    m_trg = jnp.zeros((_HF, _H), jnp.float32).at[rows, hcol].set(
        a_trg.reshape(_HF).astype(jnp.float32))
    a2 = jnp.concatenate([m_src, m_trg[:, ::-1]], axis=1)

    proj, stab = _tc_prep(xp, W_proj.astype(jnp.float32), a2)

    zo = jnp.zeros((_NPS, _HF), jnp.float32)
    partials, dens = _sc_edges(epk, stab, proj, zo)

    e_mat = jnp.concatenate([
        (hcol[None, :] == jnp.arange(_H, dtype=jnp.int32)[:, None]
         ).astype(jnp.float32),
        jnp.zeros((_H, _HF), jnp.float32),
    ], axis=0)
    out = _tc_fin(partials, dens, e_mat, xp[:_NPS],
                  W_skip.astype(jnp.float32),
                  bias.reshape(1, _HF).astype(jnp.float32))
    return out[:_N]


# speedup vs baseline: 60.1175x; 1.3102x over previous
import functools

import jax
import jax.numpy as jnp
from jax import lax
from jax.experimental import pallas as pl
from jax.experimental.pallas import tpu as pltpu
from jax.experimental.pallas import tpu_sc as plsc

_N = 10000
_E = 320000
_H = 8
_F = 16
_HF = _H * _F

_TILES = 16
_CORES = 2
_NP = 10112
_ROWS_PT = _NP // _TILES
_NPS = 10016
_RPS = _NPS // _TILES
_BLK = 128
_IC = 10
_NBLK = 80
_EP = _TILES * _CORES * _NBLK * _BLK
_SHIFT = 14
_MASK = (1 << _SHIFT) - 1


def _tc_prep_body(x_ref, wp_ref, a2_ref, proj_ref, sc_ref):
    proj = jnp.dot(x_ref[:], wp_ref[:], preferred_element_type=jnp.float32)
    proj_ref[:] = proj.astype(jnp.bfloat16)
    sc_ref[:] = jnp.dot(proj, a2_ref[:], preferred_element_type=jnp.float32)


_tc_prep = pl.pallas_call(
    _tc_prep_body,
    out_shape=[
        jax.ShapeDtypeStruct((_NP, _HF), jnp.bfloat16),
        jax.ShapeDtypeStruct((_NP, _F), jnp.float32),
    ],
)


_mesh = plsc.VectorSubcoreMesh(core_axis_name="c", subcore_axis_name="s")


def _edge_p(a_row, t_row):
    lane = lax.iota(jnp.int32, 16)
    sel = jnp.where(lane < _H, a_row, t_row)
    sco = sel + lax.rev(sel, dimensions=(0,))
    sco = jnp.maximum(sco, 0.2 * sco)
    return jnp.exp(sco)


@functools.partial(
    pl.kernel,
    out_type=[
        jax.ShapeDtypeStruct((_CORES, _NPS, _HF), jnp.bfloat16),
        jax.ShapeDtypeStruct((_CORES, _NPS, _F), jnp.float32),
    ],
    mesh=_mesh,
    compiler_params=pltpu.CompilerParams(
        use_tc_tiling_on_sc=False, needs_layout_passes=False),
    scratch_types=[
        pltpu.VMEM((_IC * _BLK,), jnp.int32),
        pltpu.VMEM((_BLK,), jnp.int32),
        pltpu.VMEM((_BLK,), jnp.int32),
        pltpu.VMEM((_BLK,), jnp.int32),
        pltpu.VMEM((_BLK,), jnp.int32),
        pltpu.VMEM((_BLK, _F), jnp.float32),
        pltpu.VMEM((_BLK, _F), jnp.float32),
        pltpu.VMEM((_BLK, _F), jnp.float32),
        pltpu.VMEM((_BLK, _F), jnp.float32),
        pltpu.VMEM((_BLK, _F), jnp.float32),
        pltpu.VMEM((_BLK, _F), jnp.float32),
        pltpu.VMEM((_BLK, _HF), jnp.bfloat16),
        pltpu.SemaphoreType.DMA,
        pltpu.SemaphoreType.DMA,
        pltpu.SemaphoreType.DMA,
        pltpu.SemaphoreType.DMA,
        pltpu.SemaphoreType.DMA,
        pltpu.VMEM_SHARED((_NPS, _HF), jnp.bfloat16),
        pltpu.VMEM_SHARED((_NPS, _F), jnp.float32),
    ],
)
def _sc_edges(epk_hbm, stab_hbm, proj_hbm, zo_hbm, zd_hbm,
              out_hbm, den_hbm, pk_i, si0, si1, ti0, ti1, sa0, sa1,
              st0, st1, at0, at1, pj, g0, g1, psem, s0, s1,
              out_sp, den_sp):
    c = lax.axis_index("c")
    s = lax.axis_index("s")
    w = s * _CORES + c
    r0 = s * _RPS
    e0 = w * (_NBLK * _BLK)

    src_i = (si0, si1)
    trg_i = (ti0, ti1)
    sa_v = (sa0, sa1)
    st_v = (st0, st1)
    att_v = (at0, at1)
    gsem = (g0, g1)
    ssem = (s0, s1)

    def load_chunk(j):
        pltpu.sync_copy(
            epk_hbm.at[pl.ds(e0 + j * _BLK, _IC * _BLK)], pk_i)

    def unpack(toff, b):
        @pl.loop(0, _BLK // 16)
        def _u(q):
            v = pk_i[pl.ds(toff * _BLK + q * 16, 16)]
            src_i[b][pl.ds(q * 16, 16)] = jnp.bitwise_and(v, _MASK)
            trg_i[b][pl.ds(q * 16, 16)] = jnp.right_shift(v, _SHIFT)

    def fire_g(b):
        pltpu.async_copy(stab_hbm.at[src_i[b]], sa_v[b], gsem[b])
        pltpu.async_copy(stab_hbm.at[trg_i[b]], st_v[b], gsem[b])

    def wait_g(b):
        pltpu.make_async_copy(stab_hbm.at[src_i[b]], sa_v[b], gsem[b]).wait()
        pltpu.make_async_copy(stab_hbm.at[trg_i[b]], st_v[b], gsem[b]).wait()

    def fire_pj(b):
        pltpu.async_copy(proj_hbm.at[src_i[b]], pj, psem)

    def wait_pj(b):
        pltpu.make_async_copy(proj_hbm.at[src_i[b]], pj, psem).wait()

    def fire_s(b):
        pltpu.async_copy(att_v[b], den_sp.at[trg_i[b]], ssem[b], add=True)
        pltpu.async_copy(pj, out_sp.at[trg_i[b]], ssem[b], add=True)

    def wait_s(b):
        pltpu.make_async_copy(att_v[b], den_sp.at[trg_i[b]], ssem[b]).wait()
        pltpu.make_async_copy(
            pj, out_sp.at[trg_i[b]], ssem[b]).wait()

    def compute(b):
        lane = lax.iota(jnp.int32, 16)

        @pl.loop(0, _BLK)
        def _edge(r):
            p = _edge_p(sa_v[b][r, :], st_v[b][r, :])
            att_v[b][r, :] = p
            for q in range(_HF // 32):
                g32 = pj[r, pl.ds(q * 32, 32)]
                ga, gb = plsc.unpack(g32, format=plsc.PackFormat.INTERLEAVED)
                sc_pair = jnp.where(lane < 8, p[2 * q], p[2 * q + 1])
                pj[r, pl.ds(q * 32, 32)] = plsc.pack(
                    ga * sc_pair, gb * sc_pair,
                    format=plsc.PackFormat.INTERLEAVED)

    pltpu.sync_copy(zo_hbm.at[pl.ds(r0, _RPS)],
                    out_sp.at[pl.ds(r0, _RPS)])
    pltpu.sync_copy(zd_hbm.at[pl.ds(r0, _RPS)],
                    den_sp.at[pl.ds(r0, _RPS)])
    load_chunk(0)
    unpack(0, 0)
    fire_g(0)
    plsc.subcore_barrier()

    @pl.loop(0, _NBLK // 2)
    def _pair(k):
        for b in (0, 1):
            j = 2 * k + b
            wait_g(b)

            @pl.when(j >= 1)
            def _():
                wait_s(1 - b)

            fire_pj(b)

            @pl.when(j < _NBLK - 1)
            def _():
                jn = j + 1
                tn = lax.rem(jn, _IC)

                @pl.when(tn == 0)
                def _():
                    load_chunk(jn)

                unpack(tn, 1 - b)
                fire_g(1 - b)

            wait_pj(b)
            compute(b)
            fire_s(b)

    wait_s(1)
    plsc.subcore_barrier()
    pltpu.sync_copy(out_sp.at[pl.ds(r0, _RPS)],
                    out_hbm.at[c, pl.ds(r0, _RPS)])
    pltpu.sync_copy(den_sp.at[pl.ds(r0, _RPS)],
                    den_hbm.at[c, pl.ds(r0, _RPS)])


def _tc_fin_body(p_ref, d_ref, e_ref, x_ref, ws_ref, b_ref, o_ref):
    agg = (p_ref[0].astype(jnp.float32) + p_ref[1].astype(jnp.float32))
    den = d_ref[0] + d_ref[1]
    recip = 1.0 / (den + 1e-16)
    recip128 = jnp.dot(recip, e_ref[:], preferred_element_type=jnp.float32)
    acc = agg * recip128
    acc = acc + jnp.dot(x_ref[:], ws_ref[:], preferred_element_type=jnp.float32)
    acc = acc + b_ref[:]
    o_ref[:] = jnp.where(acc > 0, acc, jnp.exp(jnp.minimum(acc, 0.0)) - 1.0)


_tc_fin = pl.pallas_call(
    _tc_fin_body,
    out_shape=jax.ShapeDtypeStruct((_NPS, _HF), jnp.float32),
)


def kernel(node_features, edge_index, W_proj, a_src, a_trg, W_skip, bias):
    x = node_features.astype(jnp.float32)
    xp = jnp.pad(x, ((0, _NP - _N), (0, 0)))

    src = edge_index[0].astype(jnp.int32)
    trg = edge_index[1].astype(jnp.int32)
    pad_e = _EP - _E
    src_p = jnp.concatenate([src, jnp.full((pad_e,), _N, jnp.int32)])
    trg_p = jnp.concatenate([trg, jnp.full((pad_e,), _N, jnp.int32)])
    epk = src_p + (trg_p << _SHIFT)

    rows = jnp.arange(_HF, dtype=jnp.int32)
    hcol = rows // _F
    m_src = jnp.zeros((_HF, _H), jnp.float32).at[rows, hcol].set(
        a_src.reshape(_HF).astype(jnp.float32))
    m_trg = jnp.zeros((_HF, _H), jnp.float32).at[rows, hcol].set(
        a_trg.reshape(_HF).astype(jnp.float32))
    a2 = jnp.concatenate([m_src, m_trg[:, ::-1]], axis=1)

    proj, stab = _tc_prep(xp, W_proj.astype(jnp.float32), a2)

    zo = jnp.zeros((_NPS, _HF), jnp.bfloat16)
    zd = jnp.zeros((_NPS, _F), jnp.float32)
    partials, dens = _sc_edges(epk, stab, proj, zo, zd)

    e_mat = jnp.concatenate([
        (hcol[None, :] == jnp.arange(_H, dtype=jnp.int32)[:, None]
         ).astype(jnp.float32),
        jnp.zeros((_H, _HF), jnp.float32),
    ], axis=0)
    out = _tc_fin(partials, dens, e_mat, xp[:_NPS],
                  W_skip.astype(jnp.float32),
                  bias.reshape(1, _HF).astype(jnp.float32))
    return out[:_N]


# speedup vs baseline: 87.6036x; 1.4572x over previous
import functools

import jax
import jax.numpy as jnp
from jax import lax
from jax.experimental import pallas as pl
from jax.experimental.pallas import tpu as pltpu
from jax.experimental.pallas import tpu_sc as plsc

_N = 10000
_E = 320000
_H = 8
_F = 16
_HF = _H * _F

_TILES = 16
_CORES = 2
_NP = 10112
_ROWS_PT = _NP // _TILES
_NPS = 10016
_RPS = _NPS // _TILES
_BLK = 128
_IC = 10
_NBLK = 80
_EP = _TILES * _CORES * _NBLK * _BLK
_SHIFT = 14
_MASK = (1 << _SHIFT) - 1


def _tc_prep_body(x_ref, wp_ref, a2_ref, proj_ref, sc_ref):
    proj = jnp.dot(x_ref[:], wp_ref[:], preferred_element_type=jnp.float32)
    proj_ref[:] = proj.astype(jnp.bfloat16)
    sc_ref[:] = jnp.dot(proj, a2_ref[:], preferred_element_type=jnp.float32)


_tc_prep = pl.pallas_call(
    _tc_prep_body,
    out_shape=[
        jax.ShapeDtypeStruct((_NP, _HF), jnp.bfloat16),
        jax.ShapeDtypeStruct((_NP, _F), jnp.float32),
    ],
)


_mesh = plsc.VectorSubcoreMesh(core_axis_name="c", subcore_axis_name="s")


def _edge_p(a_row, t_row):
    lane = lax.iota(jnp.int32, 16)
    sel = jnp.where(lane < _H, a_row, t_row)
    sco = sel + lax.rev(sel, dimensions=(0,))
    sco = jnp.maximum(sco, 0.2 * sco)
    return jnp.exp(sco)


@functools.partial(
    pl.kernel,
    out_type=[
        jax.ShapeDtypeStruct((_CORES, _NPS, _HF), jnp.bfloat16),
        jax.ShapeDtypeStruct((_CORES, _NPS, _F), jnp.float32),
    ],
    mesh=_mesh,
    compiler_params=pltpu.CompilerParams(
        use_tc_tiling_on_sc=False, needs_layout_passes=False),
    scratch_types=[
        pltpu.VMEM((_IC * _BLK,), jnp.int32),
        pltpu.VMEM((_BLK,), jnp.int32),
        pltpu.VMEM((_BLK,), jnp.int32),
        pltpu.VMEM((_BLK,), jnp.int32),
        pltpu.VMEM((_BLK,), jnp.int32),
        pltpu.VMEM((_BLK, _F), jnp.float32),
        pltpu.VMEM((_BLK, _F), jnp.float32),
        pltpu.VMEM((_BLK, _F), jnp.float32),
        pltpu.VMEM((_BLK, _F), jnp.float32),
        pltpu.VMEM((_BLK, _F), jnp.float32),
        pltpu.VMEM((_BLK, _F), jnp.float32),
        pltpu.VMEM((_BLK, _HF), jnp.bfloat16),
        pltpu.VMEM((_BLK, _HF), jnp.bfloat16),
        pltpu.SemaphoreType.DMA,
        pltpu.SemaphoreType.DMA,
        pltpu.SemaphoreType.DMA,
        pltpu.SemaphoreType.DMA,
        pltpu.VMEM_SHARED((_NPS, _HF), jnp.bfloat16),
        pltpu.VMEM_SHARED((_NPS, _F), jnp.float32),
    ],
)
def _sc_edges(epk_hbm, stab_hbm, proj_hbm, zo_hbm, zd_hbm,
              out_hbm, den_hbm, pk_i, si0, si1, ti0, ti1, sa0, sa1,
              st0, st1, at0, at1, pj0, pj1, g0, g1, s0, s1,
              out_sp, den_sp):
    c = lax.axis_index("c")
    s = lax.axis_index("s")
    w = s * _CORES + c
    r0 = s * _RPS
    e0 = w * (_NBLK * _BLK)

    src_i = (si0, si1)
    trg_i = (ti0, ti1)
    sa_v = (sa0, sa1)
    st_v = (st0, st1)
    att_v = (at0, at1)
    proj_v = (pj0, pj1)
    gsem = (g0, g1)
    ssem = (s0, s1)

    def load_chunk(j):
        pltpu.sync_copy(
            epk_hbm.at[pl.ds(e0 + j * _BLK, _IC * _BLK)], pk_i)

    def unpack(toff, b):
        @pl.loop(0, _BLK // 16)
        def _u(q):
            v = pk_i[pl.ds(toff * _BLK + q * 16, 16)]
            src_i[b][pl.ds(q * 16, 16)] = jnp.bitwise_and(v, _MASK)
            trg_i[b][pl.ds(q * 16, 16)] = jnp.right_shift(v, _SHIFT)

    def fire_g(b):
        pltpu.async_copy(stab_hbm.at[src_i[b]], sa_v[b], gsem[b])
        pltpu.async_copy(stab_hbm.at[trg_i[b]], st_v[b], gsem[b])
        pltpu.async_copy(proj_hbm.at[src_i[b]], proj_v[b], gsem[b])

    def wait_g(b):
        pltpu.make_async_copy(stab_hbm.at[src_i[b]], sa_v[b], gsem[b]).wait()
        pltpu.make_async_copy(stab_hbm.at[trg_i[b]], st_v[b], gsem[b]).wait()
        pltpu.make_async_copy(
            proj_hbm.at[src_i[b]], proj_v[b], gsem[b]).wait()

    def fire_s(b):
        pltpu.async_copy(att_v[b], den_sp.at[trg_i[b]], ssem[b], add=True)
        pltpu.async_copy(proj_v[b], out_sp.at[trg_i[b]], ssem[b], add=True)

    def wait_s(b):
        pltpu.make_async_copy(att_v[b], den_sp.at[trg_i[b]], ssem[b]).wait()
        pltpu.make_async_copy(
            proj_v[b], out_sp.at[trg_i[b]], ssem[b]).wait()

    def compute(b):
        lane = lax.iota(jnp.int32, 16)

        @pl.loop(0, _BLK)
        def _edge(r):
            p = _edge_p(sa_v[b][r, :], st_v[b][r, :])
            att_v[b][r, :] = p
            for q in range(_HF // 32):
                g32 = proj_v[b][r, pl.ds(q * 32, 32)]
                ga, gb = plsc.unpack(g32, format=plsc.PackFormat.INTERLEAVED)
                sc_pair = jnp.where(lane < 8, p[2 * q], p[2 * q + 1])
                proj_v[b][r, pl.ds(q * 32, 32)] = plsc.pack(
                    ga * sc_pair, gb * sc_pair,
                    format=plsc.PackFormat.INTERLEAVED)

    pltpu.sync_copy(zo_hbm.at[pl.ds(r0, _RPS)],
                    out_sp.at[pl.ds(r0, _RPS)])
    pltpu.sync_copy(zd_hbm.at[pl.ds(r0, _RPS)],
                    den_sp.at[pl.ds(r0, _RPS)])
    load_chunk(0)
    unpack(0, 0)
    fire_g(0)
    plsc.subcore_barrier()

    @pl.loop(0, _NBLK // 2)
    def _pair(k):
        for b in (0, 1):
            j = 2 * k + b
            wait_g(b)

            @pl.when(j >= 1)
            def _():
                wait_s(1 - b)

            @pl.when(j < _NBLK - 1)
            def _():
                jn = j + 1
                tn = lax.rem(jn, _IC)

                @pl.when(tn == 0)
                def _():
                    load_chunk(jn)

                unpack(tn, 1 - b)
                fire_g(1 - b)

            compute(b)
            fire_s(b)

    wait_s(1)
    plsc.subcore_barrier()
    pltpu.sync_copy(out_sp.at[pl.ds(r0, _RPS)],
                    out_hbm.at[c, pl.ds(r0, _RPS)])
    pltpu.sync_copy(den_sp.at[pl.ds(r0, _RPS)],
                    den_hbm.at[c, pl.ds(r0, _RPS)])


def _tc_fin_body(p_ref, d_ref, e_ref, x_ref, ws_ref, b_ref, o_ref):
    agg = (p_ref[0].astype(jnp.float32) + p_ref[1].astype(jnp.float32))
    den = d_ref[0] + d_ref[1]
    recip = 1.0 / (den + 1e-16)
    recip128 = jnp.dot(recip, e_ref[:], preferred_element_type=jnp.float32)
    acc = agg * recip128
    acc = acc + jnp.dot(x_ref[:], ws_ref[:], preferred_element_type=jnp.float32)
    acc = acc + b_ref[:]
    o_ref[:] = jnp.where(acc > 0, acc, jnp.exp(jnp.minimum(acc, 0.0)) - 1.0)


_tc_fin = pl.pallas_call(
    _tc_fin_body,
    out_shape=jax.ShapeDtypeStruct((_NPS, _HF), jnp.float32),
)


def kernel(node_features, edge_index, W_proj, a_src, a_trg, W_skip, bias):
    x = node_features.astype(jnp.float32)
    xp = jnp.pad(x, ((0, _NP - _N), (0, 0)))

    src = edge_index[0].astype(jnp.int32)
    trg = edge_index[1].astype(jnp.int32)
    pad_e = _EP - _E
    src_p = jnp.concatenate([src, jnp.full((pad_e,), _N, jnp.int32)])
    trg_p = jnp.concatenate([trg, jnp.full((pad_e,), _N, jnp.int32)])
    epk = src_p + (trg_p << _SHIFT)

    rows = jnp.arange(_HF, dtype=jnp.int32)
    hcol = rows // _F
    m_src = jnp.zeros((_HF, _H), jnp.float32).at[rows, hcol].set(
        a_src.reshape(_HF).astype(jnp.float32))
    m_trg = jnp.zeros((_HF, _H), jnp.float32).at[rows, hcol].set(
        a_trg.reshape(_HF).astype(jnp.float32))
    a2 = jnp.concatenate([m_src, m_trg[:, ::-1]], axis=1)

    proj, stab = _tc_prep(xp, W_proj.astype(jnp.float32), a2)

    zo = jnp.zeros((_NPS, _HF), jnp.bfloat16)
    zd = jnp.zeros((_NPS, _F), jnp.float32)
    partials, dens = _sc_edges(epk, stab, proj, zo, zd)

    e_mat = jnp.concatenate([
        (hcol[None, :] == jnp.arange(_H, dtype=jnp.int32)[:, None]
         ).astype(jnp.float32),
        jnp.zeros((_H, _HF), jnp.float32),
    ], axis=0)
    out = _tc_fin(partials, dens, e_mat, xp[:_NPS],
                  W_skip.astype(jnp.float32),
                  bias.reshape(1, _HF).astype(jnp.float32))
    return out[:_N]
